# Initial kernel scaffold; baseline (speedup 1.0000x reference)
import functools
import math

import jax
import jax.numpy as jnp
from jax import lax
from jax.experimental import pallas as pl
from jax.experimental.pallas import tpu as pltpu
from jax.experimental.pallas import tpu_sc as plsc

N_NODES = 10000
N_EDGES = 160000
D = 256
N_HEADS = 8
HEAD_DIM = 32

NC = 2
NS = 16
NW = NC * NS
LANES = 16

EDGES_PER_WORKER = N_EDGES // NW
P1_CHUNK = 200
EDGES_PER_SUBCORE = N_EDGES // NS
P3_CHUNK = 200
ROWS_PER_SUBCORE = N_NODES // NS
HALF = D // NC


def _sc_mesh():
    return plsc.VectorSubcoreMesh(
        core_axis_name="c", subcore_axis_name="s", num_cores=NC, num_subcores=NS
    )


def _score_body(q_hbm, k_hbm, ei_hbm, s_hbm, qr, kr, srcv, dstv, sbuf, sem):
    c = lax.axis_index("c")
    s = lax.axis_index("s")
    wid = s * NC + c
    base0 = wid * EDGES_PER_WORKER

    def chunk(i, carry):
        base = base0 + i * P1_CHUNK
        pltpu.sync_copy(ei_hbm.at[0, pl.ds(base, P1_CHUNK)], srcv)
        pltpu.sync_copy(ei_hbm.at[1, pl.ds(base, P1_CHUNK)], dstv)
        cq = pltpu.async_copy(q_hbm.at[srcv], qr, sem)
        ck = pltpu.async_copy(k_hbm.at[dstv], kr, sem)
        cq.wait()
        ck.wait()

        def edge(e, carry2):
            for h in range(N_HEADS):
                a = qr[e, pl.ds(HEAD_DIM * h, LANES)] * kr[e, pl.ds(HEAD_DIM * h, LANES)]
                b = (qr[e, pl.ds(HEAD_DIM * h + LANES, LANES)]
                     * kr[e, pl.ds(HEAD_DIM * h + LANES, LANES)])
                sbuf[h, e] = jnp.sum(a + b)
            return carry2

        lax.fori_loop(0, P1_CHUNK, edge, 0)
        for h in range(N_HEADS):
            pltpu.sync_copy(sbuf.at[h], s_hbm.at[h, pl.ds(base, P1_CHUNK)])
        return carry

    lax.fori_loop(0, EDGES_PER_WORKER // P1_CHUNK, chunk, 0)


@jax.jit
def _sc_scores(q, k, ei):
    fn = pl.kernel(
        _score_body,
        out_type=jax.ShapeDtypeStruct((N_HEADS, N_EDGES), jnp.float32),
        mesh=_sc_mesh(),
        scratch_types=[
            pltpu.VMEM((P1_CHUNK, D), jnp.float32),
            pltpu.VMEM((P1_CHUNK, D), jnp.float32),
            pltpu.VMEM((P1_CHUNK,), jnp.int32),
            pltpu.VMEM((P1_CHUNK,), jnp.int32),
            pltpu.VMEM((N_HEADS, P1_CHUNK), jnp.float32),
            pltpu.SemaphoreType.DMA,
        ],
    )
    return fn(q, k, ei)


def _agg_body(v_hbm, attn_hbm, ei_hbm, z_hbm, agg_hbm,
              acc, vr, srcv, dstv, attv, sem):
    c = lax.axis_index("c")
    s = lax.axis_index("s")
    r0 = s * ROWS_PER_SUBCORE
    pltpu.sync_copy(z_hbm.at[pl.ds(r0, ROWS_PER_SUBCORE)],
                    acc.at[pl.ds(r0, ROWS_PER_SUBCORE)])
    plsc.subcore_barrier()

    def chunk(i, carry):
        base = s * EDGES_PER_SUBCORE + i * P3_CHUNK
        pltpu.sync_copy(ei_hbm.at[0, pl.ds(base, P3_CHUNK)], srcv)
        pltpu.sync_copy(ei_hbm.at[1, pl.ds(base, P3_CHUNK)], dstv)
        pltpu.async_copy(v_hbm.at[c].at[srcv], vr, sem).wait()
        for j in range(N_HEADS // NC):
            pltpu.sync_copy(attn_hbm.at[(N_HEADS // NC) * c + j, pl.ds(base, P3_CHUNK)],
                            attv.at[j])

        def edge(e, carry2):
            for j in range(N_HEADS // NC):
                w = attv[j, e]
                for t in range(HEAD_DIM // LANES):
                    sl = pl.ds(HEAD_DIM * j + LANES * t, LANES)
                    vr[e, sl] = vr[e, sl] * w
            return carry2

        lax.fori_loop(0, P3_CHUNK, edge, 0)
        pltpu.sync_copy(vr, acc.at[dstv], add=True)
        return carry

    lax.fori_loop(0, EDGES_PER_SUBCORE // P3_CHUNK, chunk, 0)
    plsc.subcore_barrier()
    pltpu.sync_copy(acc.at[pl.ds(r0, ROWS_PER_SUBCORE)],
                    agg_hbm.at[c].at[pl.ds(r0, ROWS_PER_SUBCORE)])


@jax.jit
def _sc_aggregate(v, attn, ei, zeros_half):
    fn = pl.kernel(
        _agg_body,
        out_type=jax.ShapeDtypeStruct((NC, N_NODES, HALF), jnp.float32),
        mesh=_sc_mesh(),
        scratch_types=[
            pltpu.VMEM_SHARED((N_NODES, HALF), jnp.float32),
            pltpu.VMEM((P3_CHUNK, HALF), jnp.float32),
            pltpu.VMEM((P3_CHUNK,), jnp.int32),
            pltpu.VMEM((P3_CHUNK,), jnp.int32),
            pltpu.VMEM((N_HEADS // NC, P3_CHUNK), jnp.float32),
            pltpu.SemaphoreType.DMA,
        ],
    )
    return fn(v, attn, ei, zeros_half)


N_BLK = 1000
N_GRID = N_NODES // N_BLK


def _mm(x, w, b):
    return jnp.dot(x, w, preferred_element_type=jnp.float32) + b


def _enc_qkv_body(nf, we, be, wq, bq, wk, bk, wv, bv, x0o, qo, ko, vo):
    x0 = _mm(nf[...], we[...], be[...])
    x0o[...] = x0
    qo[...] = _mm(x0, wq[...], bq[...])
    ko[...] = _mm(x0, wk[...], bk[...])
    v = _mm(x0, wv[...], bv[...])
    vo[0] = v[:, :HALF]
    vo[1] = v[:, HALF:]


def _res_qkv_body(xp, agg, wq, bq, wk, bk, wv, bv, xo, qo, ko, vo):
    xi = jnp.maximum(xp[...] + jnp.concatenate([agg[0], agg[1]], axis=1), 0.0)
    xo[...] = xi
    qo[...] = _mm(xi, wq[...], bq[...])
    ko[...] = _mm(xi, wk[...], bk[...])
    v = _mm(xi, wv[...], bv[...])
    vo[0] = v[:, :HALF]
    vo[1] = v[:, HALF:]


def _softmax_body(so, ao):
    s = so[...] * (1.0 / math.sqrt(HEAD_DIM))
    m = jnp.max(s, axis=1, keepdims=True)
    w = jnp.exp(s - m)
    ao[...] = w / jnp.sum(w, axis=1, keepdims=True)


def _final_body(xp, agg, wo, bo, out):
    xi = jnp.maximum(xp[...] + jnp.concatenate([agg[0], agg[1]], axis=1), 0.0)
    emb = _mm(xi, wo[...], bo[...])
    nrm = jnp.maximum(
        jnp.sqrt(jnp.sum(emb * emb, axis=1, keepdims=True)), 1e-12)
    out[...] = emb / nrm


_row_spec = pl.BlockSpec((N_BLK, D), lambda i: (i, 0))
_w_spec = pl.BlockSpec((D, D), lambda i: (0, 0))
_b_spec = pl.BlockSpec((1, D), lambda i: (0, 0))
_v_spec = pl.BlockSpec((NC, N_BLK, HALF), lambda i: (0, i, 0))
_f32 = jnp.float32


@jax.jit
def _tc_enc_qkv(nf, we, be, wq, bq, wk, bk, wv, bv):
    return pl.pallas_call(
        _enc_qkv_body,
        grid=(N_GRID,),
        in_specs=[_row_spec] + [_w_spec, _b_spec] * 4,
        out_specs=[_row_spec, _row_spec, _row_spec, _v_spec],
        out_shape=[
            jax.ShapeDtypeStruct((N_NODES, D), _f32),
            jax.ShapeDtypeStruct((N_NODES, D), _f32),
            jax.ShapeDtypeStruct((N_NODES, D), _f32),
            jax.ShapeDtypeStruct((NC, N_NODES, HALF), _f32),
        ],
    )(nf, we, be, wq, bq, wk, bk, wv, bv)


@jax.jit
def _tc_res_qkv(xp, agg, wq, bq, wk, bk, wv, bv):
    return pl.pallas_call(
        _res_qkv_body,
        grid=(N_GRID,),
        in_specs=[_row_spec, _v_spec] + [_w_spec, _b_spec] * 3,
        out_specs=[_row_spec, _row_spec, _row_spec, _v_spec],
        out_shape=[
            jax.ShapeDtypeStruct((N_NODES, D), _f32),
            jax.ShapeDtypeStruct((N_NODES, D), _f32),
            jax.ShapeDtypeStruct((N_NODES, D), _f32),
            jax.ShapeDtypeStruct((NC, N_NODES, HALF), _f32),
        ],
    )(xp, agg, wq, bq, wk, bk, wv, bv)


@jax.jit
def _tc_softmax(s):
    return pl.pallas_call(
        _softmax_body,
        out_shape=jax.ShapeDtypeStruct((N_HEADS, N_EDGES), _f32),
    )(s)


@jax.jit
def _tc_final(xp, agg, wo, bo):
    return pl.pallas_call(
        _final_body,
        grid=(N_GRID,),
        in_specs=[_row_spec, _v_spec, _w_spec, _b_spec],
        out_specs=_row_spec,
        out_shape=jax.ShapeDtypeStruct((N_NODES, D), _f32),
    )(xp, agg, wo, bo)


def kernel(node_features, edge_index, params):
    ei = edge_index.astype(jnp.int32)
    b = {k: v.reshape(1, D) for k, v in params.items() if k.startswith("b")}
    zeros_half = jnp.zeros((N_NODES, HALF), jnp.float32)

    x, q, k, v = _tc_enc_qkv(
        node_features, params["W_enc"], b["b_enc"],
        params["Wq0"], b["bq0"], params["Wk0"], b["bk0"],
        params["Wv0"], b["bv0"])
    for i in range(3):
        s = _sc_scores(q, k, ei)
        attn = _tc_softmax(s)
        agg = _sc_aggregate(v, attn, ei, zeros_half)
        if i < 2:
            x, q, k, v = _tc_res_qkv(
                x, agg,
                params[f"Wq{i+1}"], b[f"bq{i+1}"],
                params[f"Wk{i+1}"], b[f"bk{i+1}"],
                params[f"Wv{i+1}"], b[f"bv{i+1}"])
    return _tc_final(x, agg, params["W_out"], b["b_out"])


# speedup vs baseline: 29.1249x; 29.1249x over previous
import math

import jax
import jax.numpy as jnp
from jax import lax
from jax.experimental import pallas as pl
from jax.experimental.pallas import tpu as pltpu
from jax.experimental.pallas import tpu_sc as plsc

N_NODES = 10000
N_EDGES = 160000
D = 256
N_HEADS = 8
HEAD_DIM = 32

NC = 2
NS = 16
NW = NC * NS
LANES = 16

EDGES_PER_WORKER = N_EDGES // NW
P1_CHUNK = 200
EDGES_PER_SUBCORE = N_EDGES // NS
P3_CHUNK = 200
HALF = D // NC
HPC = N_HEADS // NC


def _sc_mesh():
    return plsc.VectorSubcoreMesh(
        core_axis_name="c", subcore_axis_name="s", num_cores=NC, num_subcores=NS
    )


def _bcast_lane(vec, lane):
    idx = jnp.full((LANES,), lane, jnp.int32)
    return vec.at[idx].get(mode="promise_in_bounds")


def _score_body(q_hbm, k_hbm, src_hbm, dst_hbm, s_hbm, qr, kr, srcv, dstv,
                sbuf, sem):
    c = lax.axis_index("c")
    s = lax.axis_index("s")
    wid = s * NC + c
    base0 = wid * EDGES_PER_WORKER
    onehot = [
        (jax.lax.iota(jnp.int32, LANES) == h).astype(jnp.float32)
        for h in range(N_HEADS)
    ]

    def chunk(i, carry):
        base = base0 + i * P1_CHUNK
        pltpu.sync_copy(src_hbm.at[pl.ds(base, P1_CHUNK)], srcv)
        pltpu.sync_copy(dst_hbm.at[pl.ds(base, P1_CHUNK)], dstv)
        cq = pltpu.async_copy(q_hbm.at[srcv], qr, sem)
        ck = pltpu.async_copy(k_hbm.at[dstv], kr, sem)
        cq.wait()
        ck.wait()

        def edge(e, carry2):
            sv = jnp.zeros((LANES,), jnp.float32)
            for h in range(N_HEADS):
                a = qr[e, pl.ds(HEAD_DIM * h, LANES)] * kr[e, pl.ds(HEAD_DIM * h, LANES)]
                b = (qr[e, pl.ds(HEAD_DIM * h + LANES, LANES)]
                     * kr[e, pl.ds(HEAD_DIM * h + LANES, LANES)])
                t = jnp.sum(a + b)
                sv = sv + t * onehot[h]
            sbuf[pl.ds(e * LANES, LANES)] = sv
            return carry2

        lax.fori_loop(0, P1_CHUNK, edge, 0)
        pltpu.sync_copy(sbuf, s_hbm.at[pl.ds(base * LANES, P1_CHUNK * LANES)])
        return carry

    lax.fori_loop(0, EDGES_PER_WORKER // P1_CHUNK, chunk, 0)


@jax.jit
def _sc_scores(q, k, src, dst):
    fn = pl.kernel(
        _score_body,
        out_type=jax.ShapeDtypeStruct((N_EDGES * LANES,), jnp.float32),
        mesh=_sc_mesh(),
        compiler_params=pltpu.CompilerParams(needs_layout_passes=False),
        scratch_types=[
            pltpu.VMEM((P1_CHUNK, D), jnp.float32),
            pltpu.VMEM((P1_CHUNK, D), jnp.float32),
            pltpu.VMEM((P1_CHUNK,), jnp.int32),
            pltpu.VMEM((P1_CHUNK,), jnp.int32),
            pltpu.VMEM((P1_CHUNK * LANES,), jnp.float32),
            pltpu.SemaphoreType.DMA,
        ],
    )
    return fn(q, k, src, dst)


def _agg_body(v_hbm, attn_hbm, src_hbm, dst_hbm, z_hbm, agg_hbm,
              acc, vr, srcv, dstv, attv, sem):
    c = lax.axis_index("c")
    s = lax.axis_index("s")

    @pl.when(s == 0)
    def _zero():
        pltpu.sync_copy(z_hbm, acc)
    plsc.subcore_barrier()

    def chunk(i, carry):
        base = s * EDGES_PER_SUBCORE + i * P3_CHUNK
        pltpu.sync_copy(src_hbm.at[pl.ds(base, P3_CHUNK)], srcv)
        pltpu.sync_copy(dst_hbm.at[pl.ds(base, P3_CHUNK)], dstv)
        pltpu.async_copy(v_hbm.at[c].at[srcv], vr, sem).wait()
        pltpu.sync_copy(attn_hbm.at[pl.ds(base * LANES, P3_CHUNK * LANES)], attv)

        def edge(e, carry2):
            arow = attv[pl.ds(e * LANES, LANES)]
            for j in range(HPC):
                w = _bcast_lane(arow, HPC * c + j)
                for t in range(HEAD_DIM // LANES):
                    sl = pl.ds(HEAD_DIM * j + LANES * t, LANES)
                    vr[e, sl] = vr[e, sl] * w
            return carry2

        lax.fori_loop(0, P3_CHUNK, edge, 0)
        pltpu.sync_copy(vr, acc.at[dstv], add=True)
        return carry

    lax.fori_loop(0, EDGES_PER_SUBCORE // P3_CHUNK, chunk, 0)
    plsc.subcore_barrier()

    @pl.when(s == 0)
    def _writeout():
        pltpu.sync_copy(acc, agg_hbm.at[c])


@jax.jit
def _sc_aggregate(v, attn, src, dst, zeros_half):
    fn = pl.kernel(
        _agg_body,
        out_type=jax.ShapeDtypeStruct((NC, N_NODES, HALF), jnp.float32),
        mesh=_sc_mesh(),
        compiler_params=pltpu.CompilerParams(needs_layout_passes=False),
        scratch_types=[
            pltpu.VMEM_SHARED((N_NODES, HALF), jnp.float32),
            pltpu.VMEM((P3_CHUNK, HALF), jnp.float32),
            pltpu.VMEM((P3_CHUNK,), jnp.int32),
            pltpu.VMEM((P3_CHUNK,), jnp.int32),
            pltpu.VMEM((P3_CHUNK * LANES,), jnp.float32),
            pltpu.SemaphoreType.DMA,
        ],
    )
    return fn(v, attn, src, dst, zeros_half)


N_BLK = 1000
N_GRID = N_NODES // N_BLK


def _mm(x, w, b):
    return jnp.dot(x, w, preferred_element_type=jnp.float32) + b


def _enc_qkv_body(nf, we, be, wq, bq, wk, bk, wv, bv, x0o, qo, ko, vo):
    x0 = _mm(nf[...], we[...], be[...])
    x0o[...] = x0
    qo[...] = _mm(x0, wq[...], bq[...])
    ko[...] = _mm(x0, wk[...], bk[...])
    v = _mm(x0, wv[...], bv[...])
    vo[0] = v[:, :HALF]
    vo[1] = v[:, HALF:]


def _res_qkv_body(xp, agg, wq, bq, wk, bk, wv, bv, xo, qo, ko, vo):
    xi = jnp.maximum(xp[...] + jnp.concatenate([agg[0], agg[1]], axis=1), 0.0)
    xo[...] = xi
    qo[...] = _mm(xi, wq[...], bq[...])
    ko[...] = _mm(xi, wk[...], bk[...])
    v = _mm(xi, wv[...], bv[...])
    vo[0] = v[:, :HALF]
    vo[1] = v[:, HALF:]


def _softmax_body(so, ao):
    s = so[...] * (1.0 / math.sqrt(HEAD_DIM))
    m = jnp.max(s, axis=0, keepdims=True)
    for half in (64, 32, 16):
        m = jnp.maximum(m[:, :half], m[:, half:])
    mb = jnp.concatenate([m] * 8, axis=1)
    w = jnp.exp(s - mb)
    z = jnp.sum(w, axis=0, keepdims=True)
    for half in (64, 32, 16):
        z = z[:, :half] + z[:, half:]
    zb = jnp.concatenate([z] * 8, axis=1)
    ao[...] = w / zb


def _final_body(xp, agg, wo, bo, out):
    xi = jnp.maximum(xp[...] + jnp.concatenate([agg[0], agg[1]], axis=1), 0.0)
    emb = _mm(xi, wo[...], bo[...])
    nrm = jnp.maximum(
        jnp.sqrt(jnp.sum(emb * emb, axis=1, keepdims=True)), 1e-12)
    out[...] = emb / nrm


_row_spec = pl.BlockSpec((N_BLK, D), lambda i: (i, 0))
_w_spec = pl.BlockSpec((D, D), lambda i: (0, 0))
_b_spec = pl.BlockSpec((1, D), lambda i: (0, 0))
_v_spec = pl.BlockSpec((NC, N_BLK, HALF), lambda i: (0, i, 0))
_f32 = jnp.float32


@jax.jit
def _tc_enc_qkv(nf, we, be, wq, bq, wk, bk, wv, bv):
    return pl.pallas_call(
        _enc_qkv_body,
        grid=(N_GRID,),
        in_specs=[_row_spec] + [_w_spec, _b_spec] * 4,
        out_specs=[_row_spec, _row_spec, _row_spec, _v_spec],
        out_shape=[
            jax.ShapeDtypeStruct((N_NODES, D), _f32),
            jax.ShapeDtypeStruct((N_NODES, D), _f32),
            jax.ShapeDtypeStruct((N_NODES, D), _f32),
            jax.ShapeDtypeStruct((NC, N_NODES, HALF), _f32),
        ],
    )(nf, we, be, wq, bq, wk, bk, wv, bv)


@jax.jit
def _tc_res_qkv(xp, agg, wq, bq, wk, bk, wv, bv):
    return pl.pallas_call(
        _res_qkv_body,
        grid=(N_GRID,),
        in_specs=[_row_spec, _v_spec] + [_w_spec, _b_spec] * 3,
        out_specs=[_row_spec, _row_spec, _row_spec, _v_spec],
        out_shape=[
            jax.ShapeDtypeStruct((N_NODES, D), _f32),
            jax.ShapeDtypeStruct((N_NODES, D), _f32),
            jax.ShapeDtypeStruct((N_NODES, D), _f32),
            jax.ShapeDtypeStruct((NC, N_NODES, HALF), _f32),
        ],
    )(xp, agg, wq, bq, wk, bk, wv, bv)


@jax.jit
def _tc_softmax(s):
    rows = N_EDGES * LANES // 128
    out = pl.pallas_call(
        _softmax_body,
        out_shape=jax.ShapeDtypeStruct((rows, 128), _f32),
    )(s.reshape(rows, 128))
    return out.reshape(-1)


@jax.jit
def _tc_final(xp, agg, wo, bo):
    return pl.pallas_call(
        _final_body,
        grid=(N_GRID,),
        in_specs=[_row_spec, _v_spec, _w_spec, _b_spec],
        out_specs=_row_spec,
        out_shape=jax.ShapeDtypeStruct((N_NODES, D), _f32),
    )(xp, agg, wo, bo)


def kernel(node_features, edge_index, params):
    ei = edge_index.astype(jnp.int32)
    src, dst = ei[0], ei[1]
    b = {k: v.reshape(1, D) for k, v in params.items() if k.startswith("b")}
    zeros_half = jnp.zeros((N_NODES, HALF), jnp.float32)

    x, q, k, v = _tc_enc_qkv(
        node_features, params["W_enc"], b["b_enc"],
        params["Wq0"], b["bq0"], params["Wk0"], b["bk0"],
        params["Wv0"], b["bv0"])
    for i in range(3):
        s = _sc_scores(q, k, src, dst)
        attn = _tc_softmax(s)
        agg = _sc_aggregate(v, attn, src, dst, zeros_half)
        if i < 2:
            x, q, k, v = _tc_res_qkv(
                x, agg,
                params[f"Wq{i+1}"], b[f"bq{i+1}"],
                params[f"Wk{i+1}"], b[f"bk{i+1}"],
                params[f"Wv{i+1}"], b[f"bv{i+1}"])
    return _tc_final(x, agg, params["W_out"], b["b_out"])
